# baseline (speedup 1.0000x reference)
import functools

import jax
import jax.numpy as jnp
from jax import lax
from jax.experimental import pallas as pl
from jax.experimental.pallas import tpu as pltpu
from jax.experimental.pallas import tpu_sc as plsc

S, D, F, E = 2048, 768, 1536, 8
EPS = 1e-06
BALANCE_FACTOR = 1e-4
LANES = 128
BS = 256
NBR = (2 * S) // BS + E
NBS = S // BS
NBT = NBR + NBS
CAPR = NBR * BS
CAPX = CAPR + S
DUMP_BLK = CAPX // BS
CAPY = CAPX + BS
DW = D // 2


def _router_body(x_ref, nw_ref, wr_ref, bias_ref,
                 h_ref, i1_ref, i2_ref, w1_ref, w2_ref, loss_ref):
    x = x_ref[...]
    ms = jnp.mean(x * x, axis=1, keepdims=True)
    h = x * lax.rsqrt(ms + EPS) * nw_ref[...]
    nt = (((1,), (1,)), ((), ()))
    logits = lax.dot_general(h, wr_ref[...], nt,
                             preferred_element_type=jnp.float32)
    logits = logits + bias_ref[...]
    m = jnp.max(logits, axis=1, keepdims=True)
    p = jnp.exp(logits - m)
    probs = p / jnp.sum(p, axis=1, keepdims=True)
    lane = lax.broadcasted_iota(jnp.int32, probs.shape, 1)
    p1 = jnp.max(probs, axis=1, keepdims=True)
    i1 = jnp.min(jnp.where(probs == p1, lane, E - 1), axis=1, keepdims=True)
    probs2 = jnp.where(lane == i1, -1.0, probs)
    p2 = jnp.max(probs2, axis=1, keepdims=True)
    i2 = jnp.min(jnp.where(probs2 == p2, lane, E - 1), axis=1, keepdims=True)
    s = p1 + p2
    h_ref[...] = h.astype(jnp.bfloat16)
    i1_ref[...] = i1
    i2_ref[...] = i2
    w1_ref[...] = p1 / s
    w2_ref[...] = p2 / s
    load = jnp.sum(probs, axis=0, keepdims=True) / S
    tl = 1.0 / E
    ll = tl * (jnp.log(tl) - jnp.log(jnp.maximum(load, 1e-30)))
    loss_ref[...] = jnp.sum(ll, axis=1, keepdims=True) / E * BALANCE_FACTOR


def _sort_body(i1_hbm, i2_hbm, w1_hbm, w2_hbm,
               stok_hbm, sw_hbm, pos_hbm, bexp_hbm, inblk_hbm, outblk_hbm,
               ids_v, wv_v, rel_v, stok_v, sw_v, pos_v, startv_v,
               bexp_v, inblk_v, outblk_v):
    on = (lax.axis_index("c") == 0) & (lax.axis_index("s") == 0)

    @pl.when(on)
    def _():
        pltpu.sync_copy(i1_hbm, ids_v.at[pl.ds(0, S)])
        pltpu.sync_copy(i2_hbm, ids_v.at[pl.ds(S, S)])
        pltpu.sync_copy(w1_hbm, wv_v.at[pl.ds(0, S)])
        pltpu.sync_copy(w2_hbm, wv_v.at[pl.ds(S, S)])
        lane = lax.iota(jnp.int32, 16)

        def p1(i, cnts):
            v = ids_v[pl.ds(i * 16, 16)]
            relv = jnp.zeros(16, jnp.int32)
            new = []
            for e in range(E):
                mi = jnp.where(v == e, 1, 0)
                cs = plsc.cumsum(mi)
                relv = jnp.where(v == e, cnts[e] + cs - 1, relv)
                new.append(cnts[e] + jnp.sum(mi))
            rel_v[pl.ds(i * 16, 16)] = relv
            return tuple(new)

        cnts = lax.fori_loop(0, (2 * S) // 16, p1,
                             tuple(jnp.int32(0) for _ in range(E)))

        li0 = lane
        li1 = lane + 16
        bexp_v[pl.ds(0, 16)] = jnp.full(16, E, jnp.int32)
        bexp_v[pl.ds(16, 16)] = jnp.full(16, E, jnp.int32)
        inblk_v[pl.ds(0, 16)] = jnp.zeros(16, jnp.int32)
        inblk_v[pl.ds(16, 16)] = jnp.where(li1 >= NBR, li1, 0)
        outblk_v[pl.ds(0, 16)] = jnp.full(16, DUMP_BLK, jnp.int32)
        outblk_v[pl.ds(16, 16)] = jnp.where(li1 >= NBR, li1, DUMP_BLK)

        blk_off = jnp.int32(0)
        startv = jnp.zeros(16, jnp.int32)
        for e in range(E):
            nb_e = jnp.right_shift(cnts[e] + (BS - 1), 8)
            mslot = li0 < nb_e
            plsc.store_scatter(bexp_v, [blk_off + li0],
                               jnp.full(16, e, jnp.int32), mask=mslot)
            plsc.store_scatter(inblk_v, [blk_off + li0], blk_off + li0,
                               mask=mslot)
            plsc.store_scatter(outblk_v, [blk_off + li0], blk_off + li0,
                               mask=mslot)
            startv = jnp.where(lane == e, blk_off * BS, startv)
            blk_off = blk_off + nb_e
        startv_v[pl.ds(0, 16)] = startv

        def init_tok(i, _):
            stok_v[pl.ds(i * 16, 16)] = jnp.zeros(16, jnp.int32)
            return 0

        lax.fori_loop(0, CAPR // 16, init_tok, 0)

        def init_w(i, _):
            p = i * 16 + lane
            sw_v[pl.ds(i * 16, 16)] = jnp.where(
                (p >= CAPR) & (p < CAPX), 1.0, 0.0)
            return 0

        lax.fori_loop(0, CAPY // 16, init_w, 0)

        def p2(i, _):
            v = ids_v[pl.ds(i * 16, 16)]
            relv = rel_v[pl.ds(i * 16, 16)]
            offv = plsc.load_gather(startv_v, [v])
            posv = offv + relv
            j = i * 16 + lane
            tok = jnp.where(j >= S, j - S, j)
            plsc.store_scatter(stok_v, [posv], tok)
            plsc.store_scatter(sw_v, [posv], wv_v[pl.ds(i * 16, 16)])
            pos_v[pl.ds(i * 16, 16)] = posv
            return 0

        lax.fori_loop(0, (2 * S) // 16, p2, 0)

        pltpu.sync_copy(stok_v, stok_hbm)
        pltpu.sync_copy(sw_v, sw_hbm)
        pltpu.sync_copy(pos_v, pos_hbm)
        pltpu.sync_copy(bexp_v, bexp_hbm)
        pltpu.sync_copy(inblk_v, inblk_hbm)
        pltpu.sync_copy(outblk_v, outblk_hbm)


GCH = 64


def _gather_body(h_hbm, stok_hbm, xs_hbm, idx_v, rows0, rows1, sem0, sem1):
    wid = lax.axis_index("s") * 2 + lax.axis_index("c")
    per_w = CAPR // 32
    nch = per_w // GCH
    base = wid * per_w
    pltpu.sync_copy(stok_hbm.at[pl.ds(base, per_w)], idx_v)
    cp0 = pltpu.async_copy(h_hbm.at[idx_v.at[pl.ds(0, GCH)]], rows0, sem0)

    def chunk(k, _):
        even = lax.rem(k, 2) == 0
        nxt = (k + 1) * GCH

        @pl.when(k + 1 < nch)
        def _():
            @pl.when(even)
            def _():
                pltpu.async_copy(h_hbm.at[idx_v.at[pl.ds(nxt, GCH)]],
                                 rows1, sem1)

            @pl.when(jnp.logical_not(even))
            def _():
                pltpu.async_copy(h_hbm.at[idx_v.at[pl.ds(nxt, GCH)]],
                                 rows0, sem0)

        @pl.when(even)
        def _():
            pltpu.make_async_copy(h_hbm.at[idx_v.at[pl.ds(0, GCH)]],
                                  rows0, sem0).wait()
            pltpu.sync_copy(rows0, xs_hbm.at[pl.ds(base + k * GCH, GCH)])

        @pl.when(jnp.logical_not(even))
        def _():
            pltpu.make_async_copy(h_hbm.at[idx_v.at[pl.ds(0, GCH)]],
                                  rows1, sem1).wait()
            pltpu.sync_copy(rows1, xs_hbm.at[pl.ds(base + k * GCH, GCH)])
        return 0

    lax.fori_loop(0, nch, chunk, 0)


def _ffn_body(bexp_s, inblk_s, outblk_s,
              xs_ref, hb_ref, sw_ref, ew1_ref, ew3_ref, ew2_ref,
              sw1_ref, sw3_ref, sw2_ref, y_ref):
    i = pl.program_id(0)
    nt = (((1,), (1,)), ((), ()))

    def run(xb, w1, w3, w2):
        a = lax.dot_general(xb, w1, nt, preferred_element_type=jnp.float32)
        b = lax.dot_general(xb, w3, nt, preferred_element_type=jnp.float32)
        g = (a * (1.0 / (1.0 + jnp.exp(-a))) * b).astype(jnp.bfloat16)
        y = lax.dot_general(g, w2, nt, preferred_element_type=jnp.float32)
        y_ref[...] = y * sw_ref[...]

    valid = outblk_s[i] != DUMP_BLK
    routed = i < NBR

    @pl.when(valid & routed)
    def _():
        run(xs_ref[...],
            ew1_ref[0].astype(jnp.bfloat16),
            ew3_ref[0].astype(jnp.bfloat16),
            ew2_ref[0].astype(jnp.bfloat16))

    @pl.when(jnp.logical_not(routed))
    def _():
        run(hb_ref[...],
            sw1_ref[0].astype(jnp.bfloat16),
            sw3_ref[0].astype(jnp.bfloat16),
            sw2_ref[0].astype(jnp.bfloat16))


CCH = 32


def _combine_body(y_hbm, res_hbm, pos_hbm, out_hbm,
                  i1v, i2v, r1, r2, rsh, rr, sem):
    wid = lax.axis_index("s") * 2 + lax.axis_index("c")
    per_w = S // 32
    nch = per_w // CCH

    def chunk(k, _):
        tb = wid * per_w + k * CCH
        pltpu.sync_copy(pos_hbm.at[pl.ds(tb, CCH)], i1v)
        pltpu.sync_copy(pos_hbm.at[pl.ds(S + tb, CCH)], i2v)
        cp1 = pltpu.async_copy(y_hbm.at[i1v], r1, sem)
        cp2 = pltpu.async_copy(y_hbm.at[i2v], r2, sem)
        cp3 = pltpu.async_copy(y_hbm.at[pl.ds(CAPR + tb, CCH)], rsh, sem)
        cp4 = pltpu.async_copy(res_hbm.at[pl.ds(tb, CCH)], rr, sem)
        cp1.wait()
        cp2.wait()
        cp3.wait()
        cp4.wait()

        def row(r, _):
            for c in range(D // 16):
                cq = c * 16
                a = (r1[r, pl.ds(cq, 16)] + r2[r, pl.ds(cq, 16)]
                     + rsh[r, pl.ds(cq, 16)] + rr[r, pl.ds(cq, 16)])
                r1[r, pl.ds(cq, 16)] = a
            return 0

        lax.fori_loop(0, CCH, row, 0)
        pltpu.sync_copy(r1, out_hbm.at[pl.ds(tb, CCH)])
        return 0

    lax.fori_loop(0, nch, chunk, 0)


@functools.lru_cache(maxsize=None)
def _sc_kernels():
    mesh = plsc.VectorSubcoreMesh(core_axis_name="c", subcore_axis_name="s")
    sort_kernel = pl.kernel(
        _sort_body,
        name="sc_sort",
    out_type=(
        jax.ShapeDtypeStruct((CAPR,), jnp.int32),
        jax.ShapeDtypeStruct((CAPY,), jnp.float32),
        jax.ShapeDtypeStruct((2 * S,), jnp.int32),
        jax.ShapeDtypeStruct((NBT,), jnp.int32),
        jax.ShapeDtypeStruct((NBT,), jnp.int32),
        jax.ShapeDtypeStruct((NBT,), jnp.int32),
    ),
        mesh=mesh,
        compiler_params=pltpu.CompilerParams(needs_layout_passes=False),
        scratch_types=[
            pltpu.VMEM((2 * S,), jnp.int32),
            pltpu.VMEM((2 * S,), jnp.float32),
            pltpu.VMEM((2 * S,), jnp.int32),
            pltpu.VMEM((CAPR,), jnp.int32),
            pltpu.VMEM((CAPY,), jnp.float32),
            pltpu.VMEM((2 * S,), jnp.int32),
            pltpu.VMEM((16,), jnp.int32),
            pltpu.VMEM((NBT,), jnp.int32),
            pltpu.VMEM((NBT,), jnp.int32),
            pltpu.VMEM((NBT,), jnp.int32),
        ],
    )
    gather_kernel = pl.kernel(
        _gather_body,
    out_type=jax.ShapeDtypeStruct((CAPR, DW), jnp.int32),
        mesh=mesh,
        name="sc_gather",
        compiler_params=pltpu.CompilerParams(needs_layout_passes=False),
        scratch_types=[
            pltpu.VMEM((CAPR // 32,), jnp.int32),
            pltpu.VMEM((GCH, DW), jnp.int32),
            pltpu.VMEM((GCH, DW), jnp.int32),
            pltpu.SemaphoreType.DMA,
            pltpu.SemaphoreType.DMA,
        ],
    )
    combine_kernel = pl.kernel(
        _combine_body,
    out_type=jax.ShapeDtypeStruct((S, D), jnp.float32),
        mesh=mesh,
        name="sc_combine",
        compiler_params=pltpu.CompilerParams(needs_layout_passes=False),
        scratch_types=[
            pltpu.VMEM((CCH,), jnp.int32),
            pltpu.VMEM((CCH,), jnp.int32),
            pltpu.VMEM((CCH, D), jnp.float32),
            pltpu.VMEM((CCH, D), jnp.float32),
            pltpu.VMEM((CCH, D), jnp.float32),
            pltpu.VMEM((CCH, D), jnp.float32),
            pltpu.SemaphoreType.DMA,
        ],
    )
    return sort_kernel, gather_kernel, combine_kernel


@jax.jit
def kernel(hidden_states, norm_weight, router_weight, expert_bias,
           sw1, sw2, sw3, ew1, ew2, ew3):
    x = hidden_states.reshape(S, D)
    nw = norm_weight.reshape(1, D)

    h, i1, i2, w1, w2, loss = pl.pallas_call(
        _router_body,
        name="tc_router",
        out_shape=(
            jax.ShapeDtypeStruct((S, D), jnp.bfloat16),
            jax.ShapeDtypeStruct((S, 1), jnp.int32),
            jax.ShapeDtypeStruct((S, 1), jnp.int32),
            jax.ShapeDtypeStruct((S, 1), jnp.float32),
            jax.ShapeDtypeStruct((S, 1), jnp.float32),
            jax.ShapeDtypeStruct((1, 1), jnp.float32),
        ),
    )(x, nw, router_weight, expert_bias.reshape(1, E))

    sort_kernel, gather_kernel, combine_kernel = _sc_kernels()
    stok, swt, posf, bexp, inblk, outblk = sort_kernel(
        i1.reshape(S), i2.reshape(S), w1.reshape(S), w2.reshape(S))
    h2 = lax.bitcast_convert_type(h.reshape(S, DW, 2), jnp.int32)
    xsw = gather_kernel(h2, stok)
    xsb = lax.bitcast_convert_type(xsw, jnp.bfloat16).reshape(CAPR, D)

    y = pl.pallas_call(
        _ffn_body,
        name="tc_ffn",
        grid_spec=pltpu.PrefetchScalarGridSpec(
            num_scalar_prefetch=3,
            grid=(NBT,),
            in_specs=[
                pl.BlockSpec((BS, D),
                             lambda i, b, ib, ob: (jnp.minimum(ib[i], NBR - 1), 0)),
                pl.BlockSpec((BS, D),
                             lambda i, b, ib, ob: (jnp.maximum(i - NBR, 0), 0)),
                pl.BlockSpec((BS, 1), lambda i, b, ib, ob: (ob[i], 0)),
                pl.BlockSpec((1, F, D),
                             lambda i, b, ib, ob: (jnp.minimum(b[i], E - 1), 0, 0)),
                pl.BlockSpec((1, F, D),
                             lambda i, b, ib, ob: (jnp.minimum(b[i], E - 1), 0, 0)),
                pl.BlockSpec((1, D, F),
                             lambda i, b, ib, ob: (jnp.minimum(b[i], E - 1), 0, 0)),
                pl.BlockSpec((1, F, D), lambda i, b, ib, ob: (0, 0, 0)),
                pl.BlockSpec((1, F, D), lambda i, b, ib, ob: (0, 0, 0)),
                pl.BlockSpec((1, D, F), lambda i, b, ib, ob: (0, 0, 0)),
            ],
            out_specs=pl.BlockSpec((BS, D), lambda i, b, ib, ob: (ob[i], 0)),
        ),
        out_shape=jax.ShapeDtypeStruct((CAPY, D), jnp.float32),
    )(bexp, inblk, outblk, xsb, h, swt.reshape(CAPY, 1), ew1, ew3, ew2,
      sw1[None], sw3[None], sw2[None])

    out = combine_kernel(y, x, posf)
    return out.reshape(1, S, D), loss.reshape(())


# speedup vs baseline: 1.4049x; 1.4049x over previous
import functools

import jax
import jax.numpy as jnp
from jax import lax
from jax.experimental import pallas as pl
from jax.experimental.pallas import tpu as pltpu
from jax.experimental.pallas import tpu_sc as plsc

S, D, F, E = 2048, 768, 1536, 8
EPS = 1e-06
BALANCE_FACTOR = 1e-4
LANES = 128
BS = 256
NBR = (2 * S) // BS + E
NBS = S // BS
NBT = NBR + NBS
CAPR = NBR * BS
DUMP_BLK = NBR
CAPY = CAPR + BS


def _router_body(x_ref, nw_ref, wr_ref, bias_ref,
                 h_ref, i1_ref, i2_ref, w1_ref, w2_ref, loss_ref):
    x = x_ref[...]
    ms = jnp.mean(x * x, axis=1, keepdims=True)
    h = x * lax.rsqrt(ms + EPS) * nw_ref[...]
    nt = (((1,), (1,)), ((), ()))
    logits = lax.dot_general(h, wr_ref[...], nt,
                             preferred_element_type=jnp.float32)
    logits = logits + bias_ref[...]
    m = jnp.max(logits, axis=1, keepdims=True)
    p = jnp.exp(logits - m)
    probs = p / jnp.sum(p, axis=1, keepdims=True)
    lane = lax.broadcasted_iota(jnp.int32, probs.shape, 1)
    p1 = jnp.max(probs, axis=1, keepdims=True)
    i1 = jnp.min(jnp.where(probs == p1, lane, E - 1), axis=1, keepdims=True)
    probs2 = jnp.where(lane == i1, -1.0, probs)
    p2 = jnp.max(probs2, axis=1, keepdims=True)
    i2 = jnp.min(jnp.where(probs2 == p2, lane, E - 1), axis=1, keepdims=True)
    s = p1 + p2
    h_ref[...] = h
    i1_ref[...] = i1
    i2_ref[...] = i2
    w1_ref[...] = p1 / s
    w2_ref[...] = p2 / s
    load = jnp.sum(probs, axis=0, keepdims=True) / S
    tl = 1.0 / E
    ll = tl * (jnp.log(tl) - jnp.log(jnp.maximum(load, 1e-30)))
    loss_ref[...] = jnp.sum(ll, axis=1, keepdims=True) / E * BALANCE_FACTOR


def _sort_body(i1_hbm, i2_hbm, w1_hbm, w2_hbm,
               stok_hbm, sw_hbm, pos_hbm, bexp_hbm, inblk_hbm, outblk_hbm,
               ids_v, wv_v, rel_v, stok_v, sw_v, pos_v, startv_v,
               bexp_v, inblk_v, outblk_v):
    on = (lax.axis_index("c") == 0) & (lax.axis_index("s") == 0)

    @pl.when(on)
    def _():
        pltpu.sync_copy(i1_hbm, ids_v.at[pl.ds(0, S)])
        pltpu.sync_copy(i2_hbm, ids_v.at[pl.ds(S, S)])
        pltpu.sync_copy(w1_hbm, wv_v.at[pl.ds(0, S)])
        pltpu.sync_copy(w2_hbm, wv_v.at[pl.ds(S, S)])
        lane = lax.iota(jnp.int32, 16)

        def p1(i, cnts):
            v = ids_v[pl.ds(i * 16, 16)]
            relv = jnp.zeros(16, jnp.int32)
            new = []
            for e in range(E):
                mi = jnp.where(v == e, 1, 0)
                cs = plsc.cumsum(mi)
                relv = jnp.where(v == e, cnts[e] + cs - 1, relv)
                new.append(cnts[e] + jnp.sum(mi))
            rel_v[pl.ds(i * 16, 16)] = relv
            return tuple(new)

        cnts = lax.fori_loop(0, (2 * S) // 16, p1,
                             tuple(jnp.int32(0) for _ in range(E)))

        li0 = lane
        li1 = lane + 16
        bexp_v[pl.ds(0, 16)] = jnp.full(16, E, jnp.int32)
        bexp_v[pl.ds(16, 16)] = jnp.full(16, E, jnp.int32)
        inblk_v[pl.ds(0, 16)] = jnp.zeros(16, jnp.int32)
        inblk_v[pl.ds(16, 16)] = jnp.zeros(16, jnp.int32)
        outblk_v[pl.ds(0, 16)] = jnp.full(16, DUMP_BLK, jnp.int32)
        outblk_v[pl.ds(16, 16)] = jnp.full(16, DUMP_BLK, jnp.int32)

        blk_off = jnp.int32(0)
        startv = jnp.zeros(16, jnp.int32)
        for e in range(E):
            nb_e = jnp.right_shift(cnts[e] + (BS - 1), 8)
            mslot = li0 < nb_e
            plsc.store_scatter(bexp_v, [blk_off + li0],
                               jnp.full(16, e, jnp.int32), mask=mslot)
            plsc.store_scatter(inblk_v, [blk_off + li0], blk_off + li0,
                               mask=mslot)
            plsc.store_scatter(outblk_v, [blk_off + li0], blk_off + li0,
                               mask=mslot)
            startv = jnp.where(lane == e, blk_off * BS, startv)
            blk_off = blk_off + nb_e
        startv_v[pl.ds(0, 16)] = startv

        def init_tok(i, _):
            stok_v[pl.ds(i * 16, 16)] = jnp.zeros(16, jnp.int32)
            return 0

        lax.fori_loop(0, CAPR // 16, init_tok, 0)

        def init_w(i, _):
            sw_v[pl.ds(i * 16, 16)] = jnp.zeros(16, jnp.float32)
            return 0

        lax.fori_loop(0, CAPY // 16, init_w, 0)

        def p2(i, _):
            v = ids_v[pl.ds(i * 16, 16)]
            relv = rel_v[pl.ds(i * 16, 16)]
            offv = plsc.load_gather(startv_v, [v])
            posv = offv + relv
            j = i * 16 + lane
            tok = jnp.where(j >= S, j - S, j)
            plsc.store_scatter(stok_v, [posv], tok)
            plsc.store_scatter(sw_v, [posv], wv_v[pl.ds(i * 16, 16)])
            pos_v[pl.ds(i * 16, 16)] = posv
            return 0

        lax.fori_loop(0, (2 * S) // 16, p2, 0)

        pltpu.sync_copy(stok_v, stok_hbm)
        pltpu.sync_copy(sw_v, sw_hbm)
        pltpu.sync_copy(pos_v, pos_hbm)
        pltpu.sync_copy(bexp_v, bexp_hbm)
        pltpu.sync_copy(inblk_v, inblk_hbm)
        pltpu.sync_copy(outblk_v, outblk_hbm)


GCH = 64


def _gather_body(h_hbm, stok_hbm, xs_hbm, idx_v, rows0, rows1, sem0, sem1):
    wid = lax.axis_index("s") * 2 + lax.axis_index("c")
    per_w = CAPR // 32
    nch = per_w // GCH
    base = wid * per_w
    pltpu.sync_copy(stok_hbm.at[pl.ds(base, per_w)], idx_v)
    pltpu.async_copy(h_hbm.at[idx_v.at[pl.ds(0, GCH)]], rows0, sem0)

    def chunk(k, _):
        even = lax.rem(k, 2) == 0
        nxt = (k + 1) * GCH

        @pl.when(k + 1 < nch)
        def _():
            @pl.when(even)
            def _():
                pltpu.async_copy(h_hbm.at[idx_v.at[pl.ds(nxt, GCH)]],
                                 rows1, sem1)

            @pl.when(jnp.logical_not(even))
            def _():
                pltpu.async_copy(h_hbm.at[idx_v.at[pl.ds(nxt, GCH)]],
                                 rows0, sem0)

        @pl.when(even)
        def _():
            pltpu.make_async_copy(h_hbm.at[idx_v.at[pl.ds(0, GCH)]],
                                  rows0, sem0).wait()
            pltpu.sync_copy(rows0, xs_hbm.at[pl.ds(base + k * GCH, GCH)])

        @pl.when(jnp.logical_not(even))
        def _():
            pltpu.make_async_copy(h_hbm.at[idx_v.at[pl.ds(0, GCH)]],
                                  rows1, sem1).wait()
            pltpu.sync_copy(rows1, xs_hbm.at[pl.ds(base + k * GCH, GCH)])
        return 0

    lax.fori_loop(0, nch, chunk, 0)


def _ffn_body(bexp_s, inblk_s, outblk_s,
              xs_ref, sw_ref, ew1_ref, ew3_ref, ew2_ref, y_ref):
    i = pl.program_id(0)
    nt = (((1,), (1,)), ((), ()))

    @pl.when(outblk_s[i] != DUMP_BLK)
    def _():
        xb = xs_ref[...].astype(jnp.bfloat16)
        w1 = ew1_ref[0].astype(jnp.bfloat16)
        w3 = ew3_ref[0].astype(jnp.bfloat16)
        w2 = ew2_ref[0].astype(jnp.bfloat16)
        a = lax.dot_general(xb, w1, nt, preferred_element_type=jnp.float32)
        b = lax.dot_general(xb, w3, nt, preferred_element_type=jnp.float32)
        g = (a * (1.0 / (1.0 + jnp.exp(-a))) * b).astype(jnp.bfloat16)
        y = lax.dot_general(g, w2, nt, preferred_element_type=jnp.float32)
        y_ref[...] = y * sw_ref[...]


def _sffn_body(h_ref, w1_ref, w3_ref, w2_ref, y_ref):
    nt = (((1,), (1,)), ((), ()))
    xb = h_ref[...].astype(jnp.bfloat16)
    w1 = w1_ref[...].astype(jnp.bfloat16)
    w3 = w3_ref[...].astype(jnp.bfloat16)
    w2 = w2_ref[...].astype(jnp.bfloat16)
    a = lax.dot_general(xb, w1, nt, preferred_element_type=jnp.float32)
    b = lax.dot_general(xb, w3, nt, preferred_element_type=jnp.float32)
    g = (a * (1.0 / (1.0 + jnp.exp(-a))) * b).astype(jnp.bfloat16)
    y_ref[...] = lax.dot_general(g, w2, nt,
                                 preferred_element_type=jnp.float32)


CCH = 32


def _combine_body(y_hbm, ysh_hbm, res_hbm, pos_hbm, out_hbm,
                  i1v, i2v, r1, r2, rsh, rr, sem):
    wid = lax.axis_index("s") * 2 + lax.axis_index("c")
    per_w = S // 32
    nch = per_w // CCH

    def chunk(k, _):
        tb = wid * per_w + k * CCH
        pltpu.sync_copy(pos_hbm.at[pl.ds(tb, CCH)], i1v)
        pltpu.sync_copy(pos_hbm.at[pl.ds(S + tb, CCH)], i2v)
        cp1 = pltpu.async_copy(y_hbm.at[i1v], r1, sem)
        cp2 = pltpu.async_copy(y_hbm.at[i2v], r2, sem)
        cp3 = pltpu.async_copy(ysh_hbm.at[pl.ds(tb, CCH)], rsh, sem)
        cp4 = pltpu.async_copy(res_hbm.at[pl.ds(tb, CCH)], rr, sem)
        cp1.wait()
        cp2.wait()
        cp3.wait()
        cp4.wait()

        def row(r, _):
            for c in range(D // 16):
                cq = c * 16
                a = (r1[r, pl.ds(cq, 16)] + r2[r, pl.ds(cq, 16)]
                     + rsh[r, pl.ds(cq, 16)] + rr[r, pl.ds(cq, 16)])
                r1[r, pl.ds(cq, 16)] = a
            return 0

        lax.fori_loop(0, CCH, row, 0)
        pltpu.sync_copy(r1, out_hbm.at[pl.ds(tb, CCH)])
        return 0

    lax.fori_loop(0, nch, chunk, 0)


@functools.lru_cache(maxsize=None)
def _sc_kernels():
    mesh = plsc.VectorSubcoreMesh(core_axis_name="c", subcore_axis_name="s")
    sort_kernel = pl.kernel(
        _sort_body,
        name="sc_sort",
    out_type=(
        jax.ShapeDtypeStruct((CAPR,), jnp.int32),
        jax.ShapeDtypeStruct((CAPY,), jnp.float32),
        jax.ShapeDtypeStruct((2 * S,), jnp.int32),
        jax.ShapeDtypeStruct((NBT,), jnp.int32),
        jax.ShapeDtypeStruct((NBT,), jnp.int32),
        jax.ShapeDtypeStruct((NBT,), jnp.int32),
    ),
        mesh=mesh,
        compiler_params=pltpu.CompilerParams(needs_layout_passes=False),
        scratch_types=[
            pltpu.VMEM((2 * S,), jnp.int32),
            pltpu.VMEM((2 * S,), jnp.float32),
            pltpu.VMEM((2 * S,), jnp.int32),
            pltpu.VMEM((CAPR,), jnp.int32),
            pltpu.VMEM((CAPY,), jnp.float32),
            pltpu.VMEM((2 * S,), jnp.int32),
            pltpu.VMEM((16,), jnp.int32),
            pltpu.VMEM((NBT,), jnp.int32),
            pltpu.VMEM((NBT,), jnp.int32),
            pltpu.VMEM((NBT,), jnp.int32),
        ],
    )
    gather_kernel = pl.kernel(
        _gather_body,
    out_type=jax.ShapeDtypeStruct((CAPR, D), jnp.float32),
        mesh=mesh,
        name="sc_gather",
        compiler_params=pltpu.CompilerParams(needs_layout_passes=False),
        scratch_types=[
            pltpu.VMEM((CAPR // 32,), jnp.int32),
            pltpu.VMEM((GCH, D), jnp.float32),
            pltpu.VMEM((GCH, D), jnp.float32),
            pltpu.SemaphoreType.DMA,
            pltpu.SemaphoreType.DMA,
        ],
    )
    combine_kernel = pl.kernel(
        _combine_body,
    out_type=jax.ShapeDtypeStruct((S, D), jnp.float32),
        mesh=mesh,
        name="sc_combine",
        compiler_params=pltpu.CompilerParams(needs_layout_passes=False),
        scratch_types=[
            pltpu.VMEM((CCH,), jnp.int32),
            pltpu.VMEM((CCH,), jnp.int32),
            pltpu.VMEM((CCH, D), jnp.float32),
            pltpu.VMEM((CCH, D), jnp.float32),
            pltpu.VMEM((CCH, D), jnp.float32),
            pltpu.VMEM((CCH, D), jnp.float32),
            pltpu.SemaphoreType.DMA,
        ],
    )
    return sort_kernel, gather_kernel, combine_kernel


@jax.jit
def kernel(hidden_states, norm_weight, router_weight, expert_bias,
           sw1, sw2, sw3, ew1, ew2, ew3):
    x = hidden_states.reshape(S, D)
    nw = norm_weight.reshape(1, D)

    h, i1, i2, w1, w2, loss = pl.pallas_call(
        _router_body,
        name="tc_router",
        out_shape=(
            jax.ShapeDtypeStruct((S, D), jnp.float32),
            jax.ShapeDtypeStruct((S, 1), jnp.int32),
            jax.ShapeDtypeStruct((S, 1), jnp.int32),
            jax.ShapeDtypeStruct((S, 1), jnp.float32),
            jax.ShapeDtypeStruct((S, 1), jnp.float32),
            jax.ShapeDtypeStruct((1, 1), jnp.float32),
        ),
    )(x, nw, router_weight, expert_bias.reshape(1, E))

    sort_kernel, gather_kernel, combine_kernel = _sc_kernels()
    stok, swt, posf, bexp, inblk, outblk = sort_kernel(
        i1.reshape(S), i2.reshape(S), w1.reshape(S), w2.reshape(S))
    xs = gather_kernel(h, stok)

    ysh = pl.pallas_call(
        _sffn_body,
        name="tc_sffn",
        grid=(NBS,),
        in_specs=[
            pl.BlockSpec((BS, D), lambda i: (i, 0)),
            pl.BlockSpec((F, D), lambda i: (0, 0)),
            pl.BlockSpec((F, D), lambda i: (0, 0)),
            pl.BlockSpec((D, F), lambda i: (0, 0)),
        ],
        out_specs=pl.BlockSpec((BS, D), lambda i: (i, 0)),
        out_shape=jax.ShapeDtypeStruct((S, D), jnp.float32),
    )(h, sw1, sw3, sw2)

    y = pl.pallas_call(
        _ffn_body,
        name="tc_ffn",
        grid_spec=pltpu.PrefetchScalarGridSpec(
            num_scalar_prefetch=3,
            grid=(NBR,),
            in_specs=[
                pl.BlockSpec((BS, D), lambda i, b, ib, ob: (ib[i], 0)),
                pl.BlockSpec((BS, 1), lambda i, b, ib, ob: (ob[i], 0)),
                pl.BlockSpec((1, F, D),
                             lambda i, b, ib, ob: (jnp.minimum(b[i], E - 1), 0, 0)),
                pl.BlockSpec((1, F, D),
                             lambda i, b, ib, ob: (jnp.minimum(b[i], E - 1), 0, 0)),
                pl.BlockSpec((1, D, F),
                             lambda i, b, ib, ob: (jnp.minimum(b[i], E - 1), 0, 0)),
            ],
            out_specs=pl.BlockSpec((BS, D), lambda i, b, ib, ob: (ob[i], 0)),
        ),
        out_shape=jax.ShapeDtypeStruct((CAPY, D), jnp.float32),
    )(bexp, inblk, outblk, xs, swt.reshape(CAPY, 1), ew1, ew3, ew2)

    out = combine_kernel(y, ysh, x, posf)
    return out.reshape(1, S, D), loss.reshape(())


# speedup vs baseline: 1.7777x; 1.2654x over previous
import functools

import jax
import jax.numpy as jnp
from jax import lax
from jax.experimental import pallas as pl
from jax.experimental.pallas import tpu as pltpu
from jax.experimental.pallas import tpu_sc as plsc

S, D, F, E = 2048, 768, 1536, 8
EPS = 1e-06
BALANCE_FACTOR = 1e-4
LANES = 128
BS = 256
NBR = (2 * S) // BS + E
NBS = S // BS
NBT = NBR + NBS
CAPR = NBR * BS
DUMP_BLK = NBR
CAPY = CAPR + BS


def _router_body(x_ref, nw_ref, wr_ref, bias_ref,
                 h_ref, i1_ref, i2_ref, w1_ref, w2_ref, loss_ref):
    x = x_ref[...]
    ms = jnp.mean(x * x, axis=1, keepdims=True)
    h = x * lax.rsqrt(ms + EPS) * nw_ref[...]
    nt = (((1,), (1,)), ((), ()))
    logits = lax.dot_general(h, wr_ref[...], nt,
                             preferred_element_type=jnp.float32)
    logits = logits + bias_ref[...]
    m = jnp.max(logits, axis=1, keepdims=True)
    p = jnp.exp(logits - m)
    probs = p / jnp.sum(p, axis=1, keepdims=True)
    lane = lax.broadcasted_iota(jnp.int32, probs.shape, 1)
    p1 = jnp.max(probs, axis=1, keepdims=True)
    i1 = jnp.min(jnp.where(probs == p1, lane, E - 1), axis=1, keepdims=True)
    probs2 = jnp.where(lane == i1, -1.0, probs)
    p2 = jnp.max(probs2, axis=1, keepdims=True)
    i2 = jnp.min(jnp.where(probs2 == p2, lane, E - 1), axis=1, keepdims=True)
    s = p1 + p2
    h_ref[...] = h
    i1_ref[...] = i1
    i2_ref[...] = i2
    w1_ref[...] = p1 / s
    w2_ref[...] = p2 / s
    load = jnp.sum(probs, axis=0, keepdims=True) / S
    tl = 1.0 / E
    ll = tl * (jnp.log(tl) - jnp.log(jnp.maximum(load, 1e-30)))
    loss_ref[...] = jnp.sum(ll, axis=1, keepdims=True) / E * BALANCE_FACTOR


def _sort_body(i1_hbm, i2_hbm, w1_hbm, w2_hbm,
               stok_hbm, sw_hbm, pos_hbm, bexp_hbm, inblk_hbm, outblk_hbm,
               used_hbm,
               ids_v, wv_v, rel_v, stok_v, sw_v, pos_v, startv_v,
               bexp_v, inblk_v, outblk_v, used_v):
    on = (lax.axis_index("c") == 0) & (lax.axis_index("s") == 0)

    @pl.when(on)
    def _():
        pltpu.sync_copy(i1_hbm, ids_v.at[pl.ds(0, S)])
        pltpu.sync_copy(i2_hbm, ids_v.at[pl.ds(S, S)])
        pltpu.sync_copy(w1_hbm, wv_v.at[pl.ds(0, S)])
        pltpu.sync_copy(w2_hbm, wv_v.at[pl.ds(S, S)])
        lane = lax.iota(jnp.int32, 16)

        def p1(i, cnts):
            v = ids_v[pl.ds(i * 16, 16)]
            relv = jnp.zeros(16, jnp.int32)
            new = []
            for e in range(E):
                mi = jnp.where(v == e, 1, 0)
                cs = plsc.cumsum(mi)
                relv = jnp.where(v == e, cnts[e] + cs - 1, relv)
                new.append(cnts[e] + jnp.sum(mi))
            rel_v[pl.ds(i * 16, 16)] = relv
            return tuple(new)

        cnts = lax.fori_loop(0, (2 * S) // 16, p1,
                             tuple(jnp.int32(0) for _ in range(E)))

        li0 = lane
        li1 = lane + 16
        bexp_v[pl.ds(0, 16)] = jnp.full(16, E, jnp.int32)
        bexp_v[pl.ds(16, 16)] = jnp.full(16, E, jnp.int32)
        inblk_v[pl.ds(0, 16)] = jnp.zeros(16, jnp.int32)
        inblk_v[pl.ds(16, 16)] = jnp.zeros(16, jnp.int32)
        outblk_v[pl.ds(0, 16)] = jnp.full(16, DUMP_BLK, jnp.int32)
        outblk_v[pl.ds(16, 16)] = jnp.full(16, DUMP_BLK, jnp.int32)

        blk_off = jnp.int32(0)
        startv = jnp.zeros(16, jnp.int32)
        for e in range(E):
            nb_e = jnp.right_shift(cnts[e] + (BS - 1), 8)
            mslot = li0 < nb_e
            plsc.store_scatter(bexp_v, [blk_off + li0],
                               jnp.full(16, e, jnp.int32), mask=mslot)
            plsc.store_scatter(inblk_v, [blk_off + li0], blk_off + li0,
                               mask=mslot)
            plsc.store_scatter(outblk_v, [blk_off + li0], blk_off + li0,
                               mask=mslot)
            startv = jnp.where(lane == e, blk_off * BS, startv)
            blk_off = blk_off + nb_e
        startv_v[pl.ds(0, 16)] = startv
        used_v[pl.ds(0, 16)] = jnp.zeros(16, jnp.int32) + blk_off * BS

        def init_tok(i, _):
            stok_v[pl.ds(i * 16, 16)] = jnp.zeros(16, jnp.int32)
            return 0

        lax.fori_loop(0, CAPR // 16, init_tok, 0)

        def init_w(i, _):
            sw_v[pl.ds(i * 16, 16)] = jnp.zeros(16, jnp.float32)
            return 0

        lax.fori_loop(0, CAPY // 16, init_w, 0)

        def p2(i, _):
            v = ids_v[pl.ds(i * 16, 16)]
            relv = rel_v[pl.ds(i * 16, 16)]
            offv = plsc.load_gather(startv_v, [v])
            posv = offv + relv
            j = i * 16 + lane
            tok = jnp.where(j >= S, j - S, j)
            plsc.store_scatter(stok_v, [posv], tok)
            plsc.store_scatter(sw_v, [posv], wv_v[pl.ds(i * 16, 16)])
            pos_v[pl.ds(i * 16, 16)] = posv
            return 0

        lax.fori_loop(0, (2 * S) // 16, p2, 0)

        pltpu.sync_copy(stok_v, stok_hbm)
        pltpu.sync_copy(sw_v, sw_hbm)
        pltpu.sync_copy(pos_v, pos_hbm)
        pltpu.sync_copy(bexp_v, bexp_hbm)
        pltpu.sync_copy(inblk_v, inblk_hbm)
        pltpu.sync_copy(outblk_v, outblk_hbm)
        pltpu.sync_copy(used_v, used_hbm)


GCH = 64


def _gather_body(h_hbm, stok_hbm, used_hbm, xs_hbm,
                 uv, idx_v, rows0, rows1, sem0, sem1):
    wid = lax.axis_index("s") * 2 + lax.axis_index("c")
    per_w = CAPR // 32
    nch = per_w // GCH
    base = wid * per_w
    pltpu.sync_copy(used_hbm, uv)
    used = jnp.max(uv[pl.ds(0, 16)])
    nact = jnp.clip((used - base + (GCH - 1)) // GCH, 0, nch)
    pltpu.sync_copy(stok_hbm.at[pl.ds(base, per_w)], idx_v)

    @pl.when(nact > 0)
    def _():
        pltpu.async_copy(h_hbm.at[idx_v.at[pl.ds(0, GCH)]], rows0, sem0)

    def chunk(k, _):
        even = lax.rem(k, 2) == 0
        nxt = (k + 1) * GCH

        @pl.when(k + 1 < nact)
        def _():
            @pl.when(even)
            def _():
                pltpu.async_copy(h_hbm.at[idx_v.at[pl.ds(nxt, GCH)]],
                                 rows1, sem1)

            @pl.when(jnp.logical_not(even))
            def _():
                pltpu.async_copy(h_hbm.at[idx_v.at[pl.ds(nxt, GCH)]],
                                 rows0, sem0)

        @pl.when(even)
        def _():
            pltpu.make_async_copy(h_hbm.at[idx_v.at[pl.ds(0, GCH)]],
                                  rows0, sem0).wait()
            pltpu.sync_copy(rows0, xs_hbm.at[pl.ds(base + k * GCH, GCH)])

        @pl.when(jnp.logical_not(even))
        def _():
            pltpu.make_async_copy(h_hbm.at[idx_v.at[pl.ds(0, GCH)]],
                                  rows1, sem1).wait()
            pltpu.sync_copy(rows1, xs_hbm.at[pl.ds(base + k * GCH, GCH)])
        return 0

    lax.fori_loop(0, nact, chunk, 0)


def _ffn_body(bexp_s, inblk_s, outblk_s,
              xs_ref, sw_ref, ew1_ref, ew3_ref, ew2_ref, y_ref):
    i = pl.program_id(0)
    nt = (((1,), (1,)), ((), ()))

    @pl.when(outblk_s[i] != DUMP_BLK)
    def _():
        xb = xs_ref[...].astype(jnp.bfloat16)
        w1 = ew1_ref[0].astype(jnp.bfloat16)
        w3 = ew3_ref[0].astype(jnp.bfloat16)
        w2 = ew2_ref[0].astype(jnp.bfloat16)
        a = lax.dot_general(xb, w1, nt, preferred_element_type=jnp.float32)
        b = lax.dot_general(xb, w3, nt, preferred_element_type=jnp.float32)
        g = (a * (1.0 / (1.0 + jnp.exp(-a))) * b).astype(jnp.bfloat16)
        y = lax.dot_general(g, w2, nt, preferred_element_type=jnp.float32)
        y_ref[...] = y * sw_ref[...]


def _sffn_body(h_ref, x_ref, w1_ref, w3_ref, w2_ref, y_ref):
    nt = (((1,), (1,)), ((), ()))
    xb = h_ref[...].astype(jnp.bfloat16)
    w1 = w1_ref[...].astype(jnp.bfloat16)
    w3 = w3_ref[...].astype(jnp.bfloat16)
    w2 = w2_ref[...].astype(jnp.bfloat16)
    a = lax.dot_general(xb, w1, nt, preferred_element_type=jnp.float32)
    b = lax.dot_general(xb, w3, nt, preferred_element_type=jnp.float32)
    g = (a * (1.0 / (1.0 + jnp.exp(-a))) * b).astype(jnp.bfloat16)
    y = lax.dot_general(g, w2, nt, preferred_element_type=jnp.float32)
    y_ref[...] = y + x_ref[...]


CCH = 32


def _combine_body(y_hbm, ysh_hbm, pos_hbm, out_hbm,
                  i1v, i2v, r1, r2, rsh, sem):
    wid = lax.axis_index("s") * 2 + lax.axis_index("c")
    per_w = S // 32
    nch = per_w // CCH

    def chunk(k, _):
        tb = wid * per_w + k * CCH
        pltpu.sync_copy(pos_hbm.at[pl.ds(tb, CCH)], i1v)
        pltpu.sync_copy(pos_hbm.at[pl.ds(S + tb, CCH)], i2v)
        cp1 = pltpu.async_copy(y_hbm.at[i1v], r1, sem)
        cp2 = pltpu.async_copy(y_hbm.at[i2v], r2, sem)
        cp3 = pltpu.async_copy(ysh_hbm.at[pl.ds(tb, CCH)], rsh, sem)
        cp1.wait()
        cp2.wait()
        cp3.wait()

        def row(r, _):
            for c in range(D // 16):
                cq = c * 16
                a = (r1[r, pl.ds(cq, 16)] + r2[r, pl.ds(cq, 16)]
                     + rsh[r, pl.ds(cq, 16)])
                r1[r, pl.ds(cq, 16)] = a
            return 0

        lax.fori_loop(0, CCH, row, 0)
        pltpu.sync_copy(r1, out_hbm.at[pl.ds(tb, CCH)])
        return 0

    lax.fori_loop(0, nch, chunk, 0)


@functools.lru_cache(maxsize=None)
def _sc_kernels():
    mesh = plsc.VectorSubcoreMesh(core_axis_name="c", subcore_axis_name="s")
    sort_kernel = pl.kernel(
        _sort_body,
        name="sc_sort",
    out_type=(
        jax.ShapeDtypeStruct((CAPR,), jnp.int32),
        jax.ShapeDtypeStruct((CAPY,), jnp.float32),
        jax.ShapeDtypeStruct((2 * S,), jnp.int32),
        jax.ShapeDtypeStruct((NBT,), jnp.int32),
        jax.ShapeDtypeStruct((NBT,), jnp.int32),
        jax.ShapeDtypeStruct((NBT,), jnp.int32),
        jax.ShapeDtypeStruct((16,), jnp.int32),
    ),
        mesh=mesh,
        compiler_params=pltpu.CompilerParams(needs_layout_passes=False),
        scratch_types=[
            pltpu.VMEM((2 * S,), jnp.int32),
            pltpu.VMEM((2 * S,), jnp.float32),
            pltpu.VMEM((2 * S,), jnp.int32),
            pltpu.VMEM((CAPR,), jnp.int32),
            pltpu.VMEM((CAPY,), jnp.float32),
            pltpu.VMEM((2 * S,), jnp.int32),
            pltpu.VMEM((16,), jnp.int32),
            pltpu.VMEM((NBT,), jnp.int32),
            pltpu.VMEM((NBT,), jnp.int32),
            pltpu.VMEM((NBT,), jnp.int32),
            pltpu.VMEM((16,), jnp.int32),
        ],
    )
    gather_kernel = pl.kernel(
        _gather_body,
    out_type=jax.ShapeDtypeStruct((CAPR, D), jnp.float32),
        mesh=mesh,
        name="sc_gather",
        compiler_params=pltpu.CompilerParams(needs_layout_passes=False),
        scratch_types=[
            pltpu.VMEM((16,), jnp.int32),
            pltpu.VMEM((CAPR // 32,), jnp.int32),
            pltpu.VMEM((GCH, D), jnp.float32),
            pltpu.VMEM((GCH, D), jnp.float32),
            pltpu.SemaphoreType.DMA,
            pltpu.SemaphoreType.DMA,
        ],
    )
    combine_kernel = pl.kernel(
        _combine_body,
    out_type=jax.ShapeDtypeStruct((S, D), jnp.float32),
        mesh=mesh,
        name="sc_combine",
        compiler_params=pltpu.CompilerParams(needs_layout_passes=False),
        scratch_types=[
            pltpu.VMEM((CCH,), jnp.int32),
            pltpu.VMEM((CCH,), jnp.int32),
            pltpu.VMEM((CCH, D), jnp.float32),
            pltpu.VMEM((CCH, D), jnp.float32),
            pltpu.VMEM((CCH, D), jnp.float32),
            pltpu.SemaphoreType.DMA,
        ],
    )
    return sort_kernel, gather_kernel, combine_kernel


@jax.jit
def kernel(hidden_states, norm_weight, router_weight, expert_bias,
           sw1, sw2, sw3, ew1, ew2, ew3):
    x = hidden_states.reshape(S, D)
    nw = norm_weight.reshape(1, D)

    h, i1, i2, w1, w2, loss = pl.pallas_call(
        _router_body,
        name="tc_router",
        out_shape=(
            jax.ShapeDtypeStruct((S, D), jnp.float32),
            jax.ShapeDtypeStruct((S, 1), jnp.int32),
            jax.ShapeDtypeStruct((S, 1), jnp.int32),
            jax.ShapeDtypeStruct((S, 1), jnp.float32),
            jax.ShapeDtypeStruct((S, 1), jnp.float32),
            jax.ShapeDtypeStruct((1, 1), jnp.float32),
        ),
    )(x, nw, router_weight, expert_bias.reshape(1, E))

    sort_kernel, gather_kernel, combine_kernel = _sc_kernels()
    stok, swt, posf, bexp, inblk, outblk, used = sort_kernel(
        i1.reshape(S), i2.reshape(S), w1.reshape(S), w2.reshape(S))
    xs = gather_kernel(h, stok, used)

    ysh = pl.pallas_call(
        _sffn_body,
        name="tc_sffn",
        grid=(NBS,),
        in_specs=[
            pl.BlockSpec((BS, D), lambda i: (i, 0)),
            pl.BlockSpec((BS, D), lambda i: (i, 0)),
            pl.BlockSpec((F, D), lambda i: (0, 0)),
            pl.BlockSpec((F, D), lambda i: (0, 0)),
            pl.BlockSpec((D, F), lambda i: (0, 0)),
        ],
        out_specs=pl.BlockSpec((BS, D), lambda i: (i, 0)),
        out_shape=jax.ShapeDtypeStruct((S, D), jnp.float32),
    )(h, x, sw1, sw3, sw2)

    y = pl.pallas_call(
        _ffn_body,
        name="tc_ffn",
        grid_spec=pltpu.PrefetchScalarGridSpec(
            num_scalar_prefetch=3,
            grid=(NBR,),
            in_specs=[
                pl.BlockSpec((BS, D), lambda i, b, ib, ob: (ib[i], 0)),
                pl.BlockSpec((BS, 1), lambda i, b, ib, ob: (ob[i], 0)),
                pl.BlockSpec((1, F, D),
                             lambda i, b, ib, ob: (jnp.minimum(b[i], E - 1), 0, 0)),
                pl.BlockSpec((1, F, D),
                             lambda i, b, ib, ob: (jnp.minimum(b[i], E - 1), 0, 0)),
                pl.BlockSpec((1, D, F),
                             lambda i, b, ib, ob: (jnp.minimum(b[i], E - 1), 0, 0)),
            ],
            out_specs=pl.BlockSpec((BS, D), lambda i, b, ib, ob: (ob[i], 0)),
        ),
        out_shape=jax.ShapeDtypeStruct((CAPY, D), jnp.float32),
    )(bexp, inblk, outblk, xs, swt.reshape(CAPY, 1), ew1, ew3, ew2)

    out = combine_kernel(y, ysh, posf)
    return out.reshape(1, S, D), loss.reshape(())
